# Initial kernel scaffold; baseline (speedup 1.0000x reference)
import functools

import jax
import jax.numpy as jnp
from jax import lax
from jax.experimental import pallas as pl
from jax.experimental.pallas import tpu as pltpu
from jax.experimental.pallas import tpu_sc as plsc

B, L, H = 1024, 128, 128
BL = B * L
_SIZES = (119, 16, 12, 14, 14, 2, 8, 9, 2)
NF = len(_SIZES)

NC, NS = 2, 16
NW = NC * NS
ROWS_PER_W = BL // NW
C = 128
NCHUNK = ROWS_PER_W // C
HV = H // 16


def _make_sc_call():
    mesh = plsc.VectorSubcoreMesh(
        core_axis_name="c", subcore_axis_name="s", num_cores=NC, num_subcores=NS
    )
    scratch = (
        [pltpu.VMEM((n * H,), jnp.float32) for n in _SIZES]
        + [pltpu.VMEM((3 * H,), jnp.float32)]
        + [pltpu.VMEM((C,), jnp.int32) for _ in range(NF)]
        + [pltpu.VMEM((C * 3,), jnp.float32)]
        + [pltpu.VMEM((C * H,), jnp.float32)]
    )

    @functools.partial(
        pl.kernel,
        mesh=mesh,
        out_type=jax.ShapeDtypeStruct((BL * H,), jnp.float32),
        scratch_types=scratch,
    )
    def sc_kernel(*refs):
        idx_hbm = refs[0:NF]
        tab_hbm = refs[NF : 2 * NF]
        pos_hbm = refs[2 * NF]
        wp_hbm = refs[2 * NF + 1]
        out_hbm = refs[2 * NF + 2]
        tab_v = refs[2 * NF + 3 : 3 * NF + 3]
        wp_v = refs[3 * NF + 3]
        idx_v = refs[3 * NF + 4 : 4 * NF + 4]
        pos_v = refs[4 * NF + 4]
        out_v = refs[4 * NF + 5]

        wid = lax.axis_index("s") * NC + lax.axis_index("c")
        base0 = wid * ROWS_PER_W

        for f in range(NF):
            pltpu.sync_copy(tab_hbm[f], tab_v[f])
        pltpu.sync_copy(wp_hbm, wp_v)

        def chunk_body(it, carry_outer):
            base = base0 + it * C
            for f in range(NF):
                pltpu.sync_copy(idx_hbm[f].at[pl.ds(base, C)], idx_v[f])
            pltpu.sync_copy(pos_hbm.at[pl.ds(base * 3, C * 3)], pos_v)

            wp_vecs = tuple(
                wp_v[pl.ds(k * H + j * 16, 16)] for k in range(3) for j in range(HV)
            )

            def row_body(r, wp_c):
                idx = [idx_v[f][r] for f in range(NF)]
                p = [pos_v[r * 3 + k] for k in range(3)]
                pv = [jnp.full((16,), p[k], jnp.float32) for k in range(3)]
                for j in range(HV):
                    acc = pv[0] * wp_c[j]
                    acc = acc + pv[1] * wp_c[HV + j]
                    acc = acc + pv[2] * wp_c[2 * HV + j]
                    for f in range(NF):
                        acc = acc + tab_v[f][pl.ds(idx[f] * H + j * 16, 16)]
                    out_v[pl.ds(r * H + j * 16, 16)] = acc
                return wp_c

            lax.fori_loop(0, C, row_body, wp_vecs)
            pltpu.sync_copy(out_v, out_hbm.at[pl.ds(base * H, C * H)])
            return carry_outer

        lax.fori_loop(0, NCHUNK, chunk_body, 0)

    return sc_kernel


_SC_CALL = _make_sc_call()


def kernel(atomic_number, formal_charge, degree, explicit_valence,
           implicit_valence, aromatic, hybridization, total_num_H, is_in_ring,
           W_atomic_number, W_formal_charge, W_degree, W_explicit_valence,
           W_implicit_valence, W_aromatic, W_hybridization, W_total_num_H,
           W_is_in_ring, position, W_position):
    idxs = [atomic_number, formal_charge, degree, explicit_valence,
            implicit_valence, aromatic, hybridization, total_num_H, is_in_ring]
    tabs = [W_atomic_number, W_formal_charge, W_degree, W_explicit_valence,
            W_implicit_valence, W_aromatic, W_hybridization, W_total_num_H,
            W_is_in_ring]
    idxs = [i.reshape(BL).astype(jnp.int32) for i in idxs]
    tabs = [t.reshape(-1).astype(jnp.float32) for t in tabs]
    pos = position.reshape(BL * 3).astype(jnp.float32)
    wp = W_position.reshape(3 * H).astype(jnp.float32)
    out = _SC_CALL(*idxs, *tabs, pos, wp)
    return out.reshape(B, L, H)


# speedup vs baseline: 2.9301x; 2.9301x over previous
import functools

import jax
import jax.numpy as jnp
from jax import lax
from jax.experimental import pallas as pl
from jax.experimental.pallas import tpu as pltpu
from jax.experimental.pallas import tpu_sc as plsc

B, L, H = 1024, 128, 128
BL = B * L
_SIZES = (119, 16, 12, 14, 14, 2, 8, 9, 2)
NF = len(_SIZES)

NC, NS = 2, 16
NW = NC * NS
ROWS_PER_W = BL // NW
C = 128
NCHUNK = ROWS_PER_W // C
HV = H // 16


def _make_sc_call():
    mesh = plsc.VectorSubcoreMesh(
        core_axis_name="c", subcore_axis_name="s", num_cores=NC, num_subcores=NS
    )
    scratch = (
        [pltpu.VMEM((n * H,), jnp.float32) for n in _SIZES]
        + [pltpu.VMEM((3 * H,), jnp.float32)]
        + [pltpu.VMEM((C,), jnp.int32) for _ in range(NF)]
        + [pltpu.VMEM((C * 3,), jnp.float32)]
        + [pltpu.VMEM((C * H,), jnp.float32)]
    )

    @functools.partial(
        pl.kernel,
        mesh=mesh,
        out_type=jax.ShapeDtypeStruct((BL * H,), jnp.float32),
        scratch_types=scratch,
    )
    def sc_kernel(*refs):
        idx_hbm = refs[0:NF]
        tab_hbm = refs[NF : 2 * NF]
        pos_hbm = refs[2 * NF]
        wp_hbm = refs[2 * NF + 1]
        out_hbm = refs[2 * NF + 2]
        tab_v = refs[2 * NF + 3 : 3 * NF + 3]
        wp_v = refs[3 * NF + 3]
        idx_v = refs[3 * NF + 4 : 4 * NF + 4]
        pos_v = refs[4 * NF + 4]
        out_v = refs[4 * NF + 5]

        wid = lax.axis_index("s") * NC + lax.axis_index("c")
        base0 = wid * ROWS_PER_W

        for f in range(NF):
            pltpu.sync_copy(tab_hbm[f], tab_v[f])
        pltpu.sync_copy(wp_hbm, wp_v)

        def chunk_body(it, carry_outer):
            base = base0 + it * C
            for f in range(NF):
                pltpu.sync_copy(idx_hbm[f].at[pl.ds(base, C)], idx_v[f])
            pltpu.sync_copy(pos_hbm.at[pl.ds(base * 3, C * 3)], pos_v)

            wp_vecs = tuple(
                wp_v[pl.ds(k * H + j * 16, 16)] for k in range(3) for j in range(HV)
            )

            def group_body(g, wp_c):
                ivecs = [idx_v[f][pl.ds(g * 16, 16)] for f in range(NF)]
                pvecs = [pos_v[pl.ds(g * 48 + m * 16, 16)] for m in range(3)]
                for rr in range(16):
                    idx = [ivecs[f][rr] for f in range(NF)]
                    pv = [
                        jnp.full((16,), pvecs[(rr * 3 + k) // 16][(rr * 3 + k) % 16],
                                 jnp.float32)
                        for k in range(3)
                    ]
                    rowoff = (g * 16 + rr) * H
                    for j in range(HV):
                        acc = pv[0] * wp_c[j]
                        acc = acc + pv[1] * wp_c[HV + j]
                        acc = acc + pv[2] * wp_c[2 * HV + j]
                        for f in range(NF):
                            acc = acc + tab_v[f][pl.ds(idx[f] * H + j * 16, 16)]
                        out_v[pl.ds(rowoff + j * 16, 16)] = acc
                return wp_c

            lax.fori_loop(0, C // 16, group_body, wp_vecs)
            pltpu.sync_copy(out_v, out_hbm.at[pl.ds(base * H, C * H)])
            return carry_outer

        lax.fori_loop(0, NCHUNK, chunk_body, 0)

    return sc_kernel


_SC_CALL = _make_sc_call()


def kernel(atomic_number, formal_charge, degree, explicit_valence,
           implicit_valence, aromatic, hybridization, total_num_H, is_in_ring,
           W_atomic_number, W_formal_charge, W_degree, W_explicit_valence,
           W_implicit_valence, W_aromatic, W_hybridization, W_total_num_H,
           W_is_in_ring, position, W_position):
    idxs = [atomic_number, formal_charge, degree, explicit_valence,
            implicit_valence, aromatic, hybridization, total_num_H, is_in_ring]
    tabs = [W_atomic_number, W_formal_charge, W_degree, W_explicit_valence,
            W_implicit_valence, W_aromatic, W_hybridization, W_total_num_H,
            W_is_in_ring]
    idxs = [i.reshape(BL).astype(jnp.int32) for i in idxs]
    tabs = [t.reshape(-1).astype(jnp.float32) for t in tabs]
    pos = position.reshape(BL * 3).astype(jnp.float32)
    wp = W_position.reshape(3 * H).astype(jnp.float32)
    out = _SC_CALL(*idxs, *tabs, pos, wp)
    return out.reshape(B, L, H)


# speedup vs baseline: 7.5306x; 2.5701x over previous
import functools

import jax
import jax.numpy as jnp
from jax import lax
from jax.experimental import pallas as pl
from jax.experimental.pallas import tpu as pltpu
from jax.experimental.pallas import tpu_sc as plsc

B, L, H = 1024, 128, 128
BL = B * L
NF = 9
_GSIZES = (238, 192, 196, 144)
NG = len(_GSIZES)

NC, NS = 2, 16
NW = NC * NS
ROWS_PER_W = BL // NW
C = 128
NCHUNK = ROWS_PER_W // C
HV = H // 16


def _make_sc_call():
    mesh = plsc.VectorSubcoreMesh(
        core_axis_name="c", subcore_axis_name="s", num_cores=NC, num_subcores=NS
    )
    scratch = (
        [pltpu.VMEM((n * H,), jnp.float32) for n in _GSIZES]
        + [pltpu.VMEM((3 * H,), jnp.float32)]
        + [pltpu.VMEM((C,), jnp.int32) for _ in range(NF)]
        + [pltpu.VMEM((C * 3,), jnp.float32)]
        + [pltpu.VMEM((C * H,), jnp.float32)]
    )

    @functools.partial(
        pl.kernel,
        mesh=mesh,
        out_type=jax.ShapeDtypeStruct((BL * H,), jnp.float32),
        scratch_types=scratch,
    )
    def sc_kernel(*refs):
        idx_hbm = refs[0:NF]
        tab_hbm = refs[NF : NF + NG]
        pos_hbm = refs[NF + NG]
        wp_hbm = refs[NF + NG + 1]
        out_hbm = refs[NF + NG + 2]
        r0 = NF + NG + 3
        tab_v = refs[r0 : r0 + NG]
        wp_v = refs[r0 + NG]
        idx_v = refs[r0 + NG + 1 : r0 + NG + 1 + NF]
        pos_v = refs[r0 + NG + 1 + NF]
        out_v = refs[r0 + NG + 2 + NF]

        wid = lax.axis_index("s") * NC + lax.axis_index("c")
        base0 = wid * ROWS_PER_W

        for g in range(NG):
            pltpu.sync_copy(tab_hbm[g], tab_v[g])
        pltpu.sync_copy(wp_hbm, wp_v)

        def chunk_body(it, carry_outer):
            base = base0 + it * C
            for f in range(NF):
                pltpu.sync_copy(idx_hbm[f].at[pl.ds(base, C)], idx_v[f])
            pltpu.sync_copy(pos_hbm.at[pl.ds(base * 3, C * 3)], pos_v)

            wp_vecs = tuple(
                wp_v[pl.ds(k * H + j * 16, 16)] for k in range(3) for j in range(HV)
            )

            def group_body(g, wp_c):
                iv = [idx_v[f][pl.ds(g * 16, 16)] for f in range(NF)]
                cv = [
                    iv[0] * 2 + iv[5],
                    iv[1] * 12 + iv[2],
                    iv[3] * 14 + iv[4],
                    (iv[6] * 9 + iv[7]) * 2 + iv[8],
                ]
                pvecs = [pos_v[pl.ds(g * 48 + m * 16, 16)] for m in range(3)]
                for rr in range(16):
                    idx = [cv[t][rr] for t in range(NG)]
                    pv = [
                        jnp.full((16,), pvecs[(rr * 3 + k) // 16][(rr * 3 + k) % 16],
                                 jnp.float32)
                        for k in range(3)
                    ]
                    rowoff = (g * 16 + rr) * H
                    for j in range(HV):
                        t01 = (tab_v[0][pl.ds(idx[0] * H + j * 16, 16)]
                               + tab_v[1][pl.ds(idx[1] * H + j * 16, 16)])
                        t23 = (tab_v[2][pl.ds(idx[2] * H + j * 16, 16)]
                               + tab_v[3][pl.ds(idx[3] * H + j * 16, 16)])
                        pacc = (pv[0] * wp_c[j] + pv[1] * wp_c[HV + j]
                                + pv[2] * wp_c[2 * HV + j])
                        out_v[pl.ds(rowoff + j * 16, 16)] = (t01 + t23) + pacc
                return wp_c

            lax.fori_loop(0, C // 16, group_body, wp_vecs)
            pltpu.sync_copy(out_v, out_hbm.at[pl.ds(base * H, C * H)])
            return carry_outer

        lax.fori_loop(0, NCHUNK, chunk_body, 0)

    return sc_kernel


_SC_CALL = _make_sc_call()


def kernel(atomic_number, formal_charge, degree, explicit_valence,
           implicit_valence, aromatic, hybridization, total_num_H, is_in_ring,
           W_atomic_number, W_formal_charge, W_degree, W_explicit_valence,
           W_implicit_valence, W_aromatic, W_hybridization, W_total_num_H,
           W_is_in_ring, position, W_position):
    idxs = [atomic_number, formal_charge, degree, explicit_valence,
            implicit_valence, aromatic, hybridization, total_num_H, is_in_ring]
    idxs = [i.reshape(BL).astype(jnp.int32) for i in idxs]
    f32 = jnp.float32
    g0 = (W_atomic_number.astype(f32)[:, None, :]
          + W_aromatic.astype(f32)[None, :, :]).reshape(-1)
    g1 = (W_formal_charge.astype(f32)[:, None, :]
          + W_degree.astype(f32)[None, :, :]).reshape(-1)
    g2 = (W_explicit_valence.astype(f32)[:, None, :]
          + W_implicit_valence.astype(f32)[None, :, :]).reshape(-1)
    g3 = (W_hybridization.astype(f32)[:, None, None, :]
          + W_total_num_H.astype(f32)[None, :, None, :]
          + W_is_in_ring.astype(f32)[None, None, :, :]).reshape(-1)
    pos = position.reshape(BL * 3).astype(f32)
    wp = W_position.reshape(3 * H).astype(f32)
    out = _SC_CALL(*idxs, g0, g1, g2, g3, pos, wp)
    return out.reshape(B, L, H)
